# Initial kernel scaffold; baseline (speedup 1.0000x reference)
import jax
import jax.numpy as jnp
from jax.experimental import pallas as pl


def kernel(x, edge_index, batch_index, edge_attr, W0, b0, W1, b1, W2, b2, W3, b3, Wout, bout):
    raise NotImplementedError("write your pallas kernel here")


# speedup vs baseline: 1.0006x; 1.0006x over previous
import jax
import jax.numpy as jnp
from jax.experimental import pallas as pl


def _pool_matmul_kernel(pooled_ref, w_ref, b_ref, o_ref):
    o_ref[...] = pooled_ref[...] @ w_ref[...] + b_ref[...]


def _gcn_conv(x, src, dst, w, W, b, dinv):
    n = x.shape[0]
    xw = x @ W
    norm = dinv[src] * w * dinv[dst]
    agg = jax.ops.segment_sum(xw[src] * norm[:, None], dst, num_segments=n)
    agg = agg + xw * (dinv * dinv)[:, None]
    return agg + b


def kernel(x, edge_index, batch_index, edge_attr, W0, b0, W1, b1, W2, b2, W3, b3, Wout, bout):
    src = edge_index[0]
    dst = edge_index[1]
    n = x.shape[0]
    G = 64
    deg = jnp.zeros((n,), dtype=x.dtype).at[dst].add(edge_attr) + 1.0
    dinv = jax.lax.rsqrt(deg)
    h = jnp.tanh(_gcn_conv(x, src, dst, edge_attr, W0, b0, dinv))
    h = jnp.tanh(_gcn_conv(h, src, dst, edge_attr, W1, b1, dinv))
    h = jnp.tanh(_gcn_conv(h, src, dst, edge_attr, W2, b2, dinv))
    h = jnp.tanh(_gcn_conv(h, src, dst, edge_attr, W3, b3, dinv))
    counts = jax.ops.segment_sum(jnp.ones((n,), dtype=h.dtype), batch_index, num_segments=G)
    mean_pool = jax.ops.segment_sum(h, batch_index, num_segments=G) / jnp.maximum(counts, 1.0)[:, None]
    max_pool = jax.ops.segment_max(h, batch_index, num_segments=G)
    max_pool = jnp.where(counts[:, None] > 0, max_pool, 0.0)
    pooled = jnp.concatenate([max_pool, mean_pool], axis=1)
    out = pl.pallas_call(
        _pool_matmul_kernel,
        out_shape=jax.ShapeDtypeStruct((G, 1), jnp.float32),
    )(pooled, Wout, bout.reshape(1, 1))
    return out.reshape(-1)


# speedup vs baseline: 6.9252x; 6.9213x over previous
import functools

import jax
import jax.numpy as jnp
from jax import lax
from jax.experimental import pallas as pl
from jax.experimental.pallas import tpu as pltpu
from jax.experimental.pallas import tpu_sc as plsc

NN = 10000
NP = 10240
HF = 64
NG = 64
NC = 2
NS = 16
NW = NC * NS
CK = 128
ROWS_PER_TILE = NP // NS
RB = 128


def _sc_deg_body(nch, dst_hbm, w_hbm, out_hbm, dst_v, w_v, dat_v, acc_sh):
    c = lax.axis_index("c")
    s = lax.axis_index("s")
    wid = s * NC + c

    pltpu.sync_copy(dst_hbm.at[pl.ds(wid * nch, nch)], dst_v)
    pltpu.sync_copy(w_hbm.at[pl.ds(wid * nch, nch)], w_v)

    for r in range(RB):
        dat_v.at[r][pl.ds(0, 16)] = jnp.zeros((16,), jnp.float32)
    for k in range(ROWS_PER_TILE // RB):
        pltpu.sync_copy(dat_v.at[pl.ds(0, RB)],
                        acc_sh.at[pl.ds(s * ROWS_PER_TILE + k * RB, RB)])
    plsc.subcore_barrier()

    def chunk(i, carry):
        for q in range(CK // 16):
            wvec = w_v.at[i][pl.ds(q * 16, 16)]
            for j in range(16):
                bw = wvec.at[jnp.full((16,), j, jnp.int32)].get(
                    mode="promise_in_bounds")
                dat_v.at[q * 16 + j][pl.ds(0, 16)] = bw
        pltpu.sync_copy(dat_v.at[pl.ds(0, CK)], acc_sh.at[dst_v.at[i]], add=True)
        return carry

    lax.fori_loop(0, nch, chunk, 0)
    plsc.subcore_barrier()

    for k in range(ROWS_PER_TILE // RB):
        r0 = s * ROWS_PER_TILE + k * RB
        pltpu.sync_copy(acc_sh.at[pl.ds(r0, RB)], dat_v.at[pl.ds(0, RB)])
        pltpu.sync_copy(dat_v.at[pl.ds(0, RB)], out_hbm.at[c, pl.ds(r0, RB)])


def _sc_deg(dst_r, w_r):
    nch = dst_r.shape[0] // NW
    mesh = plsc.VectorSubcoreMesh(core_axis_name="c", subcore_axis_name="s")
    body = functools.partial(_sc_deg_body, nch)
    f = pl.kernel(
        body,
        out_type=jax.ShapeDtypeStruct((NC, NP, 16), jnp.float32),
        mesh=mesh,
        scratch_types=[
            pltpu.VMEM((nch, CK), jnp.int32),
            pltpu.VMEM((nch, CK), jnp.float32),
            pltpu.VMEM((CK, 16), jnp.float32),
            pltpu.VMEM_SHARED((NP, 16), jnp.float32),
        ],
    )
    return f(dst_r, w_r)


def _sc_edge_body(nch, y_hbm, src_hbm, dst_hbm, w_hbm, out_hbm,
                  src_v, dst_v, w_v, rows_v, acc_sh, gsem):
    c = lax.axis_index("c")
    s = lax.axis_index("s")
    wid = s * NC + c

    pltpu.sync_copy(src_hbm.at[pl.ds(wid * nch, nch)], src_v)
    pltpu.sync_copy(dst_hbm.at[pl.ds(wid * nch, nch)], dst_v)
    pltpu.sync_copy(w_hbm.at[pl.ds(wid * nch, nch)], w_v)

    for r in range(RB):
        row = rows_v.at[r]
        for q in range(HF // 16):
            row[pl.ds(q * 16, 16)] = jnp.zeros((16,), jnp.float32)
    for k in range(ROWS_PER_TILE // RB):
        pltpu.sync_copy(rows_v.at[pl.ds(0, RB)],
                        acc_sh.at[pl.ds(s * ROWS_PER_TILE + k * RB, RB)])
    plsc.subcore_barrier()

    def chunk(i, carry):
        pltpu.async_copy(y_hbm.at[src_v.at[i]], rows_v, gsem).wait()
        for g in range(CK // 16):
            wvec = w_v.at[i][pl.ds(g * 16, 16)]
            for j in range(16):
                row = rows_v.at[g * 16 + j]
                bw = wvec.at[jnp.full((16,), j, jnp.int32)].get(
                    mode="promise_in_bounds")
                for q in range(HF // 16):
                    row[pl.ds(q * 16, 16)] = row[pl.ds(q * 16, 16)] * bw
        pltpu.sync_copy(rows_v, acc_sh.at[dst_v.at[i]], add=True)
        return carry

    lax.fori_loop(0, nch, chunk, 0)
    plsc.subcore_barrier()

    for k in range(ROWS_PER_TILE // RB):
        r0 = s * ROWS_PER_TILE + k * RB
        pltpu.sync_copy(acc_sh.at[pl.ds(r0, RB)], rows_v.at[pl.ds(0, RB)])
        pltpu.sync_copy(rows_v.at[pl.ds(0, RB)], out_hbm.at[c, pl.ds(r0, RB)])


def _sc_edge(y, src_r, dst_r, w_r):
    nch = src_r.shape[0] // NW
    mesh = plsc.VectorSubcoreMesh(core_axis_name="c", subcore_axis_name="s")
    body = functools.partial(_sc_edge_body, nch)
    f = pl.kernel(
        body,
        out_type=jax.ShapeDtypeStruct((NC, NP, HF), jnp.float32),
        mesh=mesh,
        compiler_params=pltpu.CompilerParams(use_tc_tiling_on_sc=False),
        scratch_types=[
            pltpu.VMEM((nch, CK), jnp.int32),
            pltpu.VMEM((nch, CK), jnp.int32),
            pltpu.VMEM((nch, CK), jnp.float32),
            pltpu.VMEM((CK, HF), jnp.float32),
            pltpu.VMEM_SHARED((NP, HF), jnp.float32),
            pltpu.SemaphoreType.DMA,
        ],
    )
    return f(y, src_r, dst_r, w_r)


BLK = 1024


def _tc_prep_kernel(dacc_ref, x_ref, w0_ref, dinv_ref, xw_ref, y_ref):
    deg = dacc_ref[0, :, 0:1] + dacc_ref[1, :, 0:1] + 1.0
    dinv = lax.rsqrt(deg)
    xw = jnp.dot(x_ref[...], w0_ref[...], preferred_element_type=jnp.float32)
    dinv_ref[...] = dinv
    xw_ref[...] = xw
    y_ref[...] = xw * dinv


def _tc_prep(dacc, x, W0):
    grid = NP // BLK
    return pl.pallas_call(
        _tc_prep_kernel,
        grid=(grid,),
        in_specs=[
            pl.BlockSpec((NC, BLK, 16), lambda i: (0, i, 0)),
            pl.BlockSpec((BLK, 128), lambda i: (i, 0)),
            pl.BlockSpec((128, HF), lambda i: (0, 0)),
        ],
        out_specs=[
            pl.BlockSpec((BLK, 1), lambda i: (i, 0)),
            pl.BlockSpec((BLK, HF), lambda i: (i, 0)),
            pl.BlockSpec((BLK, HF), lambda i: (i, 0)),
        ],
        out_shape=[
            jax.ShapeDtypeStruct((NP, 1), jnp.float32),
            jax.ShapeDtypeStruct((NP, HF), jnp.float32),
            jax.ShapeDtypeStruct((NP, HF), jnp.float32),
        ],
    )(dacc, x, W0)


def _tc_post_kernel(acc_ref, xw_ref, dinv_ref, b_ref, wn_ref, xwn_ref, yn_ref):
    dinv = dinv_ref[...]
    agg = (acc_ref[0] + acc_ref[1]) * dinv + xw_ref[...] * (dinv * dinv) + b_ref[...]
    h = jnp.tanh(agg)
    xwn = jnp.dot(h, wn_ref[...], preferred_element_type=jnp.float32)
    xwn_ref[...] = xwn
    yn_ref[...] = xwn * dinv


def _tc_post(acc, xw, dinv, b, Wn):
    grid = NP // BLK
    return pl.pallas_call(
        _tc_post_kernel,
        grid=(grid,),
        in_specs=[
            pl.BlockSpec((NC, BLK, HF), lambda i: (0, i, 0)),
            pl.BlockSpec((BLK, HF), lambda i: (i, 0)),
            pl.BlockSpec((BLK, 1), lambda i: (i, 0)),
            pl.BlockSpec((1, HF), lambda i: (0, 0)),
            pl.BlockSpec((HF, HF), lambda i: (0, 0)),
        ],
        out_specs=[
            pl.BlockSpec((BLK, HF), lambda i: (i, 0)),
            pl.BlockSpec((BLK, HF), lambda i: (i, 0)),
        ],
        out_shape=[
            jax.ShapeDtypeStruct((NP, HF), jnp.float32),
            jax.ShapeDtypeStruct((NP, HF), jnp.float32),
        ],
    )(acc, xw, dinv, b.reshape(1, HF), Wn)


def _tc_final_kernel(acc_ref, xw_ref, dinv_ref, b_ref, bn_ref, bt_ref,
                     wout_ref, bout_ref, out_ref, mx_ref):
    dinv = dinv_ref[...]
    agg = (acc_ref[0] + acc_ref[1]) * dinv + xw_ref[...] * (dinv * dinv) + b_ref[...]
    h = jnp.tanh(agg)

    bt = bt_ref[...]
    gids = lax.broadcasted_iota(jnp.int32, (NG, 1), 0)
    oneh = (bt == gids).astype(jnp.float32)
    sums = jnp.dot(oneh, h, preferred_element_type=jnp.float32)
    counts = jnp.sum(oneh, axis=1, keepdims=True)
    mean_p = sums / jnp.maximum(counts, 1.0)

    bn = bn_ref[...]
    neg = jnp.float32(-jnp.inf)

    def mx_body(g, carry):
        hm = jnp.where(bn == g, h, neg)
        mx_ref[pl.ds(g, 1), :] = jnp.max(hm, axis=0, keepdims=True)
        return carry

    lax.fori_loop(0, NG, mx_body, 0)
    max_p = jnp.where(counts > 0, mx_ref[...], 0.0)

    pooled = jnp.concatenate([max_p, mean_p], axis=1)
    out_ref[...] = jnp.dot(pooled, wout_ref[...],
                           preferred_element_type=jnp.float32) + bout_ref[...]


def _tc_final(acc, xw, dinv, b, batch_n1, batch_1n, Wout, bout):
    return pl.pallas_call(
        _tc_final_kernel,
        out_shape=jax.ShapeDtypeStruct((NG, 1), jnp.float32),
        scratch_shapes=[pltpu.VMEM((NG, HF), jnp.float32)],
    )(acc, xw, dinv, b.reshape(1, HF), batch_n1, batch_1n,
      Wout, bout.reshape(1, 1))


def kernel(x, edge_index, batch_index, edge_attr, W0, b0, W1, b1, W2, b2, W3, b3, Wout, bout):
    E = edge_index.shape[1]
    ep = ((E + NW * 8 * CK - 1) // (NW * 8 * CK)) * (NW * 8 * CK)
    pad = ep - E

    src = edge_index[0].astype(jnp.int32)
    dst = edge_index[1].astype(jnp.int32)
    w = edge_attr.astype(jnp.float32)
    if pad:
        zi = jnp.zeros((pad,), jnp.int32)
        src = jnp.concatenate([src, zi])
        dst = jnp.concatenate([dst, zi])
        w = jnp.concatenate([w, jnp.zeros((pad,), jnp.float32)])
    src_r = src.reshape(ep // CK, CK)
    dst_r = dst.reshape(ep // CK, CK)
    w_r = w.reshape(ep // CK, CK)

    xp = jnp.concatenate([x.astype(jnp.float32),
                          jnp.zeros((NP - NN, x.shape[1]), jnp.float32)])

    dacc = _sc_deg(dst_r, w_r)
    dinv, xw, y = _tc_prep(dacc, xp, W0)

    acc = _sc_edge(y, src_r, dst_r, w_r)
    xw, y = _tc_post(acc, xw, dinv, b0, W1)
    acc = _sc_edge(y, src_r, dst_r, w_r)
    xw, y = _tc_post(acc, xw, dinv, b1, W2)
    acc = _sc_edge(y, src_r, dst_r, w_r)
    xw, y = _tc_post(acc, xw, dinv, b2, W3)
    acc = _sc_edge(y, src_r, dst_r, w_r)

    bi = batch_index.astype(jnp.int32)
    bi = jnp.concatenate([bi, jnp.full((NP - NN,), NG, jnp.int32)])
    out = _tc_final(acc, xw, dinv, b3, bi.reshape(NP, 1), bi.reshape(1, NP),
                    Wout, bout)
    return out.reshape(-1)


# speedup vs baseline: 7.9467x; 1.1475x over previous
import functools

import jax
import jax.numpy as jnp
from jax import lax
from jax.experimental import pallas as pl
from jax.experimental.pallas import tpu as pltpu
from jax.experimental.pallas import tpu_sc as plsc

NN = 10000
NP = 10240
HF = 64
NG = 64
NC = 2
NS = 16
NW = NC * NS
CK = 128
ROWS_PER_TILE = NP // NS
RB = 128


def _sc_deg_body(nch, dst_hbm, w_hbm, out_hbm, dst_v, w_v, dat_v, acc_sh):
    c = lax.axis_index("c")
    s = lax.axis_index("s")
    wid = s * NC + c

    pltpu.sync_copy(dst_hbm.at[pl.ds(wid * nch, nch)], dst_v)
    pltpu.sync_copy(w_hbm.at[pl.ds(wid * nch, nch)], w_v)

    for r in range(RB):
        dat_v.at[r][pl.ds(0, 16)] = jnp.zeros((16,), jnp.float32)
    for k in range(ROWS_PER_TILE // RB):
        pltpu.sync_copy(dat_v.at[pl.ds(0, RB)],
                        acc_sh.at[pl.ds(s * ROWS_PER_TILE + k * RB, RB)])
    plsc.subcore_barrier()

    def chunk(i, carry):
        for q in range(CK // 16):
            wvec = w_v.at[i][pl.ds(q * 16, 16)]
            for j in range(16):
                bw = wvec.at[jnp.full((16,), j, jnp.int32)].get(
                    mode="promise_in_bounds")
                dat_v.at[q * 16 + j][pl.ds(0, 16)] = bw
        pltpu.sync_copy(dat_v.at[pl.ds(0, CK)], acc_sh.at[dst_v.at[i]], add=True)
        return carry

    lax.fori_loop(0, nch, chunk, 0)
    plsc.subcore_barrier()

    for k in range(ROWS_PER_TILE // RB):
        r0 = s * ROWS_PER_TILE + k * RB
        pltpu.sync_copy(acc_sh.at[pl.ds(r0, RB)], dat_v.at[pl.ds(0, RB)])
        pltpu.sync_copy(dat_v.at[pl.ds(0, RB)], out_hbm.at[c, pl.ds(r0, RB)])


def _sc_deg(dst_r, w_r):
    nch = dst_r.shape[0] // NW
    mesh = plsc.VectorSubcoreMesh(core_axis_name="c", subcore_axis_name="s")
    body = functools.partial(_sc_deg_body, nch)
    f = pl.kernel(
        body,
        out_type=jax.ShapeDtypeStruct((NC, NP, 16), jnp.float32),
        mesh=mesh,
        scratch_types=[
            pltpu.VMEM((nch, CK), jnp.int32),
            pltpu.VMEM((nch, CK), jnp.float32),
            pltpu.VMEM((CK, 16), jnp.float32),
            pltpu.VMEM_SHARED((NP, 16), jnp.float32),
        ],
    )
    return f(dst_r, w_r)


def _sc_edge_body(nch, y_hbm, src_hbm, dst_hbm, w_hbm, out_hbm,
                  src_v, dst_v, w_v, rows_a, rows_b, acc_sh,
                  gsa, gsb, ssa, ssb):
    c = lax.axis_index("c")
    s = lax.axis_index("s")
    wid = s * NC + c

    pltpu.sync_copy(src_hbm.at[pl.ds(wid * nch, nch)], src_v)
    pltpu.sync_copy(dst_hbm.at[pl.ds(wid * nch, nch)], dst_v)
    pltpu.sync_copy(w_hbm.at[pl.ds(wid * nch, nch)], w_v)

    for r in range(RB):
        row = rows_a.at[r]
        for q in range(HF // 16):
            row[pl.ds(q * 16, 16)] = jnp.zeros((16,), jnp.float32)
    for k in range(ROWS_PER_TILE // RB):
        pltpu.sync_copy(rows_a.at[pl.ds(0, RB)],
                        acc_sh.at[pl.ds(s * ROWS_PER_TILE + k * RB, RB)])
    plsc.subcore_barrier()

    def scale(buf, i):
        for g in range(CK // 16):
            wvec = w_v.at[i][pl.ds(g * 16, 16)]
            for j in range(16):
                row = buf.at[g * 16 + j]
                bw = wvec.at[jnp.full((16,), j, jnp.int32)].get(
                    mode="promise_in_bounds")
                for q in range(HF // 16):
                    row[pl.ds(q * 16, 16)] = row[pl.ds(q * 16, 16)] * bw

    pltpu.async_copy(y_hbm.at[src_v.at[0]], rows_a, gsa)

    def pair(k, carry):
        i0 = 2 * k
        pltpu.make_async_copy(y_hbm.at[src_v.at[i0]], rows_a, gsa).wait()
        pltpu.async_copy(y_hbm.at[src_v.at[i0 + 1]], rows_b, gsb)
        scale(rows_a, i0)
        pltpu.sync_copy(rows_a, acc_sh.at[dst_v.at[i0]], add=True)

        pltpu.make_async_copy(y_hbm.at[src_v.at[i0]], rows_b, gsb).wait()

        @pl.when(k < nch // 2 - 1)
        def _():
            pltpu.async_copy(y_hbm.at[src_v.at[i0 + 2]], rows_a, gsa)

        scale(rows_b, i0 + 1)
        pltpu.sync_copy(rows_b, acc_sh.at[dst_v.at[i0 + 1]], add=True)
        return carry

    lax.fori_loop(0, nch // 2, pair, 0)
    plsc.subcore_barrier()

    for k in range(ROWS_PER_TILE // RB):
        r0 = s * ROWS_PER_TILE + k * RB
        pltpu.sync_copy(acc_sh.at[pl.ds(r0, RB)], rows_a.at[pl.ds(0, RB)])
        pltpu.sync_copy(rows_a.at[pl.ds(0, RB)], out_hbm.at[c, pl.ds(r0, RB)])


def _sc_edge(y, src_r, dst_r, w_r):
    nch = src_r.shape[0] // NW
    mesh = plsc.VectorSubcoreMesh(core_axis_name="c", subcore_axis_name="s")
    body = functools.partial(_sc_edge_body, nch)
    f = pl.kernel(
        body,
        out_type=jax.ShapeDtypeStruct((NC, NP, HF), jnp.float32),
        mesh=mesh,
        compiler_params=pltpu.CompilerParams(use_tc_tiling_on_sc=False),
        scratch_types=[
            pltpu.VMEM((nch, CK), jnp.int32),
            pltpu.VMEM((nch, CK), jnp.int32),
            pltpu.VMEM((nch, CK), jnp.float32),
            pltpu.VMEM((CK, HF), jnp.float32),
            pltpu.VMEM((CK, HF), jnp.float32),
            pltpu.VMEM_SHARED((NP, HF), jnp.float32),
            pltpu.SemaphoreType.DMA,
            pltpu.SemaphoreType.DMA,
            pltpu.SemaphoreType.DMA,
            pltpu.SemaphoreType.DMA,
        ],
    )
    return f(y, src_r, dst_r, w_r)


BLK = 1024


def _tc_prep_kernel(dacc_ref, x_ref, w0_ref, dinv_ref, xw_ref, y_ref):
    deg = dacc_ref[0, :, 0:1] + dacc_ref[1, :, 0:1] + 1.0
    dinv = lax.rsqrt(deg)
    xw = jnp.dot(x_ref[...], w0_ref[...], preferred_element_type=jnp.float32)
    dinv_ref[...] = dinv
    xw_ref[...] = xw
    y_ref[...] = xw * dinv


def _tc_prep(dacc, x, W0):
    grid = NP // BLK
    return pl.pallas_call(
        _tc_prep_kernel,
        grid=(grid,),
        in_specs=[
            pl.BlockSpec((NC, BLK, 16), lambda i: (0, i, 0)),
            pl.BlockSpec((BLK, 128), lambda i: (i, 0)),
            pl.BlockSpec((128, HF), lambda i: (0, 0)),
        ],
        out_specs=[
            pl.BlockSpec((BLK, 1), lambda i: (i, 0)),
            pl.BlockSpec((BLK, HF), lambda i: (i, 0)),
            pl.BlockSpec((BLK, HF), lambda i: (i, 0)),
        ],
        out_shape=[
            jax.ShapeDtypeStruct((NP, 1), jnp.float32),
            jax.ShapeDtypeStruct((NP, HF), jnp.float32),
            jax.ShapeDtypeStruct((NP, HF), jnp.float32),
        ],
    )(dacc, x, W0)


def _tc_post_kernel(acc_ref, xw_ref, dinv_ref, b_ref, wn_ref, xwn_ref, yn_ref):
    dinv = dinv_ref[...]
    agg = (acc_ref[0] + acc_ref[1]) * dinv + xw_ref[...] * (dinv * dinv) + b_ref[...]
    h = jnp.tanh(agg)
    xwn = jnp.dot(h, wn_ref[...], preferred_element_type=jnp.float32)
    xwn_ref[...] = xwn
    yn_ref[...] = xwn * dinv


def _tc_post(acc, xw, dinv, b, Wn):
    grid = NP // BLK
    return pl.pallas_call(
        _tc_post_kernel,
        grid=(grid,),
        in_specs=[
            pl.BlockSpec((NC, BLK, HF), lambda i: (0, i, 0)),
            pl.BlockSpec((BLK, HF), lambda i: (i, 0)),
            pl.BlockSpec((BLK, 1), lambda i: (i, 0)),
            pl.BlockSpec((1, HF), lambda i: (0, 0)),
            pl.BlockSpec((HF, HF), lambda i: (0, 0)),
        ],
        out_specs=[
            pl.BlockSpec((BLK, HF), lambda i: (i, 0)),
            pl.BlockSpec((BLK, HF), lambda i: (i, 0)),
        ],
        out_shape=[
            jax.ShapeDtypeStruct((NP, HF), jnp.float32),
            jax.ShapeDtypeStruct((NP, HF), jnp.float32),
        ],
    )(acc, xw, dinv, b.reshape(1, HF), Wn)


def _tc_final_kernel(acc_ref, xw_ref, dinv_ref, b_ref, bn_ref, bt_ref,
                     wout_ref, bout_ref, out_ref, mx_ref):
    dinv = dinv_ref[...]
    agg = (acc_ref[0] + acc_ref[1]) * dinv + xw_ref[...] * (dinv * dinv) + b_ref[...]
    h = jnp.tanh(agg)

    bt = bt_ref[...]
    gids = lax.broadcasted_iota(jnp.int32, (NG, 1), 0)
    oneh = (bt == gids).astype(jnp.float32)
    sums = jnp.dot(oneh, h, preferred_element_type=jnp.float32)
    counts = jnp.sum(oneh, axis=1, keepdims=True)
    mean_p = sums / jnp.maximum(counts, 1.0)

    bn = bn_ref[...]
    neg = jnp.float32(-jnp.inf)

    def mx_body(g, carry):
        hm = jnp.where(bn == g, h, neg)
        mx_ref[pl.ds(g, 1), :] = jnp.max(hm, axis=0, keepdims=True)
        return carry

    lax.fori_loop(0, NG, mx_body, 0)
    max_p = jnp.where(counts > 0, mx_ref[...], 0.0)

    pooled = jnp.concatenate([max_p, mean_p], axis=1)
    out_ref[...] = jnp.dot(pooled, wout_ref[...],
                           preferred_element_type=jnp.float32) + bout_ref[...]


def _tc_final(acc, xw, dinv, b, batch_n1, batch_1n, Wout, bout):
    return pl.pallas_call(
        _tc_final_kernel,
        out_shape=jax.ShapeDtypeStruct((NG, 1), jnp.float32),
        scratch_shapes=[pltpu.VMEM((NG, HF), jnp.float32)],
    )(acc, xw, dinv, b.reshape(1, HF), batch_n1, batch_1n,
      Wout, bout.reshape(1, 1))


def kernel(x, edge_index, batch_index, edge_attr, W0, b0, W1, b1, W2, b2, W3, b3, Wout, bout):
    E = edge_index.shape[1]
    ep = ((E + NW * 8 * CK - 1) // (NW * 8 * CK)) * (NW * 8 * CK)
    pad = ep - E

    src = edge_index[0].astype(jnp.int32)
    dst = edge_index[1].astype(jnp.int32)
    w = edge_attr.astype(jnp.float32)
    if pad:
        zi = jnp.zeros((pad,), jnp.int32)
        src = jnp.concatenate([src, zi])
        dst = jnp.concatenate([dst, zi])
        w = jnp.concatenate([w, jnp.zeros((pad,), jnp.float32)])
    src_r = src.reshape(ep // CK, CK)
    dst_r = dst.reshape(ep // CK, CK)
    w_r = w.reshape(ep // CK, CK)

    xp = jnp.concatenate([x.astype(jnp.float32),
                          jnp.zeros((NP - NN, x.shape[1]), jnp.float32)])

    dacc = _sc_deg(dst_r, w_r)
    dinv, xw, y = _tc_prep(dacc, xp, W0)

    acc = _sc_edge(y, src_r, dst_r, w_r)
    xw, y = _tc_post(acc, xw, dinv, b0, W1)
    acc = _sc_edge(y, src_r, dst_r, w_r)
    xw, y = _tc_post(acc, xw, dinv, b1, W2)
    acc = _sc_edge(y, src_r, dst_r, w_r)
    xw, y = _tc_post(acc, xw, dinv, b2, W3)
    acc = _sc_edge(y, src_r, dst_r, w_r)

    bi = batch_index.astype(jnp.int32)
    bi = jnp.concatenate([bi, jnp.full((NP - NN,), NG, jnp.int32)])
    out = _tc_final(acc, xw, dinv, b3, bi.reshape(NP, 1), bi.reshape(1, NP),
                    Wout, bout)
    return out.reshape(-1)
